# Initial kernel scaffold; baseline (speedup 1.0000x reference)
import jax
import jax.numpy as jnp
from jax.experimental import pallas as pl


def kernel(x, edge_index1, edge_index2, W1_l, W1_r, b1, W2_l, W2_r, b2):
    raise NotImplementedError("write your pallas kernel here")


# speedup vs baseline: 5.4360x; 5.4360x over previous
import functools

import jax
import jax.numpy as jnp
from jax import lax
from jax.experimental import pallas as pl
from jax.experimental.pallas import tpu as pltpu
from jax.experimental.pallas import tpu_sc as plsc

N_NODES = 10000
NPAD = 10240
NC, NS = 2, 16
NW = NC * NS
CH = 128
ROWS_PER_TILE = NPAD // NS


def _prep_edges(edge_index):
    src, dst = edge_index[0], edge_index[1]
    e = src.shape[0]
    e_pad = -(-e // (NW * CH)) * (NW * CH)
    pad = e_pad - e
    src = jnp.concatenate([src, jnp.zeros((pad,), jnp.int32)])
    dst = jnp.concatenate([dst, jnp.full((pad,), N_NODES, jnp.int32)])
    n_ch = e_pad // (NW * CH)
    return src.reshape(NW, n_ch, CH), dst.reshape(NW, n_ch, CH), n_ch


def _segsum(table, src3, dst3, n_ch, width):
    mesh = plsc.VectorSubcoreMesh(core_axis_name="c", subcore_axis_name="s")
    zeros = jnp.zeros((ROWS_PER_TILE, width), jnp.float32)

    @functools.partial(
        pl.kernel,
        mesh=mesh,
        compiler_params=pltpu.CompilerParams(use_tc_tiling_on_sc=False),
        out_type=jax.ShapeDtypeStruct((NC, NPAD, width), jnp.float32),
        scratch_types=[
            pltpu.VMEM((n_ch, CH), jnp.int32),
            pltpu.VMEM((n_ch, CH), jnp.int32),
            pltpu.VMEM((CH, width), jnp.float32),
            pltpu.VMEM_SHARED((NPAD, width), jnp.float32),
            pltpu.SemaphoreType.DMA,
        ],
    )
    def k(table_hbm, src_hbm, dst_hbm, z_hbm, out_hbm, srcv, dstv, rows_v, acc, sem):
        cid = lax.axis_index("c")
        sid = lax.axis_index("s")
        wid = sid * NC + cid
        pltpu.sync_copy(src_hbm.at[wid], srcv)
        pltpu.sync_copy(dst_hbm.at[wid], dstv)
        row0 = sid * ROWS_PER_TILE
        pltpu.sync_copy(z_hbm, acc.at[pl.ds(row0, ROWS_PER_TILE)])
        plsc.subcore_barrier()

        @pl.loop(0, n_ch)
        def _(ci):
            pltpu.async_copy(table_hbm.at[srcv.at[ci]], rows_v, sem).wait()
            pltpu.sync_copy(rows_v, acc.at[dstv.at[ci]], add=True)

        plsc.subcore_barrier()
        pltpu.sync_copy(acc.at[pl.ds(row0, ROWS_PER_TILE)],
                        out_hbm.at[cid, pl.ds(row0, ROWS_PER_TILE)])

    return k(table, src3, dst3, zeros)


def _tc_table1(x, w):
    def body(x_ref, w_ref, o_ref):
        y = jnp.dot(x_ref[...], w_ref[...], preferred_element_type=jnp.float32)
        yp = jnp.pad(y, ((0, NPAD - N_NODES), (0, 16)))
        cols = lax.broadcasted_iota(jnp.int32, (NPAD, 144), 1)
        o_ref[...] = jnp.where(cols == 128, 1.0, yp)

    return pl.pallas_call(
        body, out_shape=jax.ShapeDtypeStruct((NPAD, 144), jnp.float32))(x, w)


def _tc_mid(p1, x, w1r, b1, w2l, w2r):
    def body(p_ref, x_ref, wr_ref, b_ref, wl2_ref, wr2_ref, t2_ref, hr_ref):
        p = p_ref[0] + p_ref[1]
        agg = p[:N_NODES, :128]
        cnt = jnp.maximum(p[:N_NODES, 128:129], 1.0)
        h = agg / cnt + b_ref[...] + jnp.dot(
            x_ref[...], wr_ref[...], preferred_element_type=jnp.float32)
        h = jnp.maximum(h, 0.0)
        y2 = jnp.dot(h, wl2_ref[...], preferred_element_type=jnp.float32)
        y2p = jnp.pad(y2, ((0, NPAD - N_NODES), (0, 16)))
        cols = lax.broadcasted_iota(jnp.int32, (NPAD, 80), 1)
        t2_ref[...] = jnp.where(cols == 64, 1.0, y2p)
        hr_ref[...] = jnp.dot(h, wr2_ref[...], preferred_element_type=jnp.float32)

    return pl.pallas_call(
        body,
        out_shape=[jax.ShapeDtypeStruct((NPAD, 80), jnp.float32),
                   jax.ShapeDtypeStruct((N_NODES, 64), jnp.float32)],
    )(p1, x, w1r, b1.reshape(1, -1), w2l, w2r)


def _tc_out(p2, hr, b2):
    def body(p_ref, hr_ref, b_ref, o_ref):
        p = p_ref[0] + p_ref[1]
        agg = p[:N_NODES, :64]
        cnt = jnp.maximum(p[:N_NODES, 64:65], 1.0)
        o_ref[...] = agg / cnt + b_ref[...] + hr_ref[...]

    return pl.pallas_call(
        body, out_shape=jax.ShapeDtypeStruct((N_NODES, 64), jnp.float32))(
            p2, hr, b2.reshape(1, -1))


def kernel(x, edge_index1, edge_index2, W1_l, W1_r, b1, W2_l, W2_r, b2):
    src1, dst1, n1 = _prep_edges(edge_index1)
    src2, dst2, n2 = _prep_edges(edge_index2)
    t1 = _tc_table1(x, W1_l)
    p1 = _segsum(t1, src1, dst1, n1, 144)
    t2, hr = _tc_mid(p1, x, W1_r, b1, W2_l, W2_r)
    p2 = _segsum(t2, src2, dst2, n2, 80)
    return _tc_out(p2, hr, b2)
